# baseline (device time: 49049 ns/iter reference)
import math

import jax
import jax.numpy as jnp
from jax import lax
from jax.experimental import pallas as pl
from jax.experimental.pallas import tpu as pltpu

N_DEV = 4
CHUNK = 512
N_SLOTS = 12


def kernel(q, k, v):
    s_per, d = q.shape
    qscale = math.log2(math.e) / math.sqrt(d)

    init_sends = [
        (0, 0, True), (2, 6, False),
        (1, 1, True), (3, 7, False),
        (2, 2, True), (0, 8, False),
        (3, 3, True), (1, 9, False),
    ]
    recv_plan = [
        [(0, (4, True)), (6, (10, False))],
        [(1, (5, True)), (7, (11, False))],
        [(2, None), (8, None)],
        [(3, None), (9, None)],
        [(4, None), (10, None)],
        [(5, None), (11, None)],
    ]

    def body(q_ref, k_ref, v_ref, out_ref, *scratch):
        qs_ref = scratch[0]
        l_ref = scratch[1]
        stages = scratch[2:6]
        comms = scratch[6:6 + N_SLOTS]
        send_sems = scratch[6 + N_SLOTS]
        recv_sems = scratch[7 + N_SLOTS]

        my_pos = lax.axis_index("i")
        left = (my_pos - 1) % N_DEV
        right = (my_pos + 1) % N_DEV

        qs_ref[:, :] = (q_ref[:, :] * qscale).astype(jnp.bfloat16)
        for m in range(4):
            rows = pl.ds(m * CHUNK, CHUNK)
            stages[m][pl.ds(0, CHUNK), :] = k_ref[rows, :].astype(jnp.bfloat16)
            stages[m][pl.ds(CHUNK, CHUNK), :] = v_ref[rows, :].astype(jnp.bfloat16)

        barrier_sem = pltpu.get_barrier_semaphore()
        for nbr in [left, right]:
            pl.semaphore_signal(
                barrier_sem, inc=1,
                device_id=(nbr,), device_id_type=pl.DeviceIdType.MESH,
            )
        pl.semaphore_wait(barrier_sem, 2)

        started = {}

        def send(src_ref, slot, to_right):
            rdma = pltpu.make_async_remote_copy(
                src_ref=src_ref,
                dst_ref=comms[slot],
                send_sem=send_sems.at[slot],
                recv_sem=recv_sems.at[slot],
                device_id=(right if to_right else left,),
                device_id_type=pl.DeviceIdType.MESH,
            )
            rdma.start()
            started[slot] = rdma

        for stage_idx, slot, to_right in init_sends:
            send(stages[stage_idx], slot, to_right)

        ones_blk = jnp.ones((CHUNK, 128), jnp.bfloat16)
        qs = qs_ref[:, :]

        def compute(buf, first=False):
            kb = buf[pl.ds(0, CHUNK), :]
            vb = buf[pl.ds(CHUNK, CHUNK), :]
            s = lax.dot_general(
                qs, kb, (((1,), (1,)), ((), ())),
                preferred_element_type=jnp.float32,
            )
            p = jnp.exp2(s).astype(jnp.bfloat16)
            vext = jnp.concatenate([vb, ones_blk], axis=1)
            pvx = lax.dot_general(
                p, vext, (((1,), (0,)), ((), ())),
                preferred_element_type=jnp.float32,
            )
            if first:
                out_ref[:, :] = pvx[:, :d]
                l_ref[:, :] = pvx[:, d:d + 1]
            else:
                out_ref[:, :] = out_ref[:, :] + pvx[:, :d]
                l_ref[:, :] = l_ref[:, :] + pvx[:, d:d + 1]

        for m in range(4):
            compute(stages[m], first=(m == 0))

        for pair in recv_plan:
            for slot, fwd in pair:
                started[slot].wait_recv()
                if fwd is not None:
                    send(comms[slot], fwd[0], fwd[1])
            for slot, _ in pair:
                compute(comms[slot])

        out_ref[:, :] = out_ref[:, :] / l_ref[:, :]

        for rdma in started.values():
            rdma.wait_send()

    return pl.pallas_call(
        body,
        out_shape=jax.ShapeDtypeStruct((s_per, d), jnp.float32),
        in_specs=[pl.BlockSpec(memory_space=pltpu.VMEM)] * 3,
        out_specs=pl.BlockSpec(memory_space=pltpu.VMEM),
        scratch_shapes=(
            [
                pltpu.VMEM((s_per, d), jnp.bfloat16),
                pltpu.VMEM((s_per, 1), jnp.float32),
            ]
            + [pltpu.VMEM((2 * CHUNK, d), jnp.bfloat16)] * 4
            + [pltpu.VMEM((2 * CHUNK, d), jnp.bfloat16)] * N_SLOTS
            + [
                pltpu.SemaphoreType.DMA((N_SLOTS,)),
                pltpu.SemaphoreType.DMA((N_SLOTS,)),
            ]
        ),
        compiler_params=pltpu.CompilerParams(collective_id=0),
    )(q, k, v)
